# Initial kernel scaffold; baseline (speedup 1.0000x reference)
import jax
import jax.numpy as jnp
from jax.experimental import pallas as pl


def kernel(inputs_embeds, attention_mask):
    raise NotImplementedError("write your pallas kernel here")


# speedup vs baseline: 1.6872x; 1.6872x over previous
import functools

import jax
import jax.numpy as jnp
import numpy as np
from jax import lax
from jax.experimental import pallas as pl
from jax.experimental.pallas import tpu as pltpu
from jax.experimental.pallas import tpu_sc as plsc

D_MODEL = 1024
SEQ_LEN = 2048
NUM_BLOCKS = 4
ZERO_OFFSET = 2 * SEQ_LEN
TBL_BASE = 2048
TBL_ROWS = 4160

NC, NS = 2, 16
NW = NC * NS
CHUNK = 32


def _pool_pos(pos, block_index):
    cls_pos = np.array([-(2 ** block_index) + 1], dtype=np.int64)
    return np.concatenate([cls_pos, pos[1:-1][::2]], axis=0)


def _rel_pos(pos, stride, pooled_pos=None, shift=1):
    if pooled_pos is None:
        pooled_pos = pos
    ref_point = int(pooled_pos[0]) - int(pos[0])
    num_remove = shift * len(pooled_pos)
    max_dist = ref_point + num_remove * stride
    min_dist = int(pooled_pos[0]) - int(pos[-1])
    return np.arange(max_dist, min_dist - 1, -stride, dtype=np.int64)


def _index_lists():
    pos = np.arange(0, SEQ_LEN, dtype=np.int64)
    pooled_pos = pos
    per_block = []
    for block_index in range(NUM_BLOCKS):
        pool_idx = None
        if block_index > 0:
            pooled_pos = _pool_pos(pos, block_index)
            stride = 2 ** (block_index - 1)
            pool_idx = _rel_pos(pos, stride, pooled_pos, shift=2) + ZERO_OFFSET
        pos = pooled_pos
        stride = 2 ** block_index
        np_idx = _rel_pos(pos, stride) + ZERO_OFFSET
        per_block.append((np_idx, pool_idx))
    flat = []
    for np_idx, pool_idx in per_block:
        flat.append(np_idx)
        if pool_idx is not None:
            flat.append(pool_idx)
    return [np.asarray(ix - TBL_BASE, dtype=np.int32) for ix in flat]


_IDX = _index_lists()
_SIZES = [len(ix) for ix in _IDX]
_ROWS_W = [s // NW for s in _SIZES]
_OFF_W = list(np.cumsum([0] + _ROWS_W))
_IDX_VMEM = int(np.ceil(_OFF_W[-1] / 16) * 16)


_TBL_BLK = 64
_HALF = D_MODEL // 2


def _table_body(out_ref):
    pid = pl.program_id(0)
    rel = (jnp.float32(pid * _TBL_BLK - TBL_BASE)
           + lax.broadcasted_iota(jnp.int32, (_TBL_BLK, _HALF), 0).astype(jnp.float32))
    j = lax.broadcasted_iota(jnp.int32, (_TBL_BLK, _HALF), 1).astype(jnp.float32)
    inv_freq = jnp.exp(j * jnp.float32(-np.log(10000.0) / _HALF))
    x = rel * inv_freq
    out_ref[:, :_HALF] = jnp.sin(x)
    out_ref[:, _HALF:] = jnp.cos(x)


def _build_table():
    return pl.pallas_call(
        _table_body,
        grid=(TBL_ROWS // _TBL_BLK,),
        out_specs=pl.BlockSpec((_TBL_BLK, D_MODEL), lambda i: (i, 0)),
        out_shape=jax.ShapeDtypeStruct((TBL_ROWS, D_MODEL), jnp.float32),
    )()


_CLS_BLK = 256


def _cls_body(out_ref):
    pid = pl.program_id(0)
    i = pid * _CLS_BLK + lax.broadcasted_iota(jnp.int32, (_CLS_BLK, SEQ_LEN), 0)
    j = lax.broadcasted_iota(jnp.int32, (_CLS_BLK, SEQ_LEN), 1)
    out_ref[...] = jnp.where((i > 0) & (j > 0), 1.0, 0.0).astype(jnp.float32)


def _build_cls_mask():
    return pl.pallas_call(
        _cls_body,
        grid=(SEQ_LEN // _CLS_BLK,),
        out_specs=pl.BlockSpec((_CLS_BLK, SEQ_LEN), lambda i: (i, 0)),
        out_shape=jax.ShapeDtypeStruct((SEQ_LEN, SEQ_LEN), jnp.float32),
    )()


def _sc_gather_body(table_hbm, *refs):
    idx_hbm = refs[:7]
    outs = refs[7:14]
    idx_v, buf0, buf1, gsem0, gsem1, ssem0, ssem1 = refs[14:]
    bufs = (buf0, buf1)
    gsems = (gsem0, gsem1)
    ssems = (ssem0, ssem1)

    wid = lax.axis_index("s") * NC + lax.axis_index("c")

    for o in range(7):
        rw = _ROWS_W[o]
        pltpu.sync_copy(idx_hbm[o].at[pl.ds(wid * rw, rw)],
                        idx_v.at[pl.ds(_OFF_W[o], rw)])

    pending = [None, None]
    k = 0
    for o in range(7):
        rw = _ROWS_W[o]
        base = wid * rw
        for c0 in range(0, rw, CHUNK):
            c = min(CHUNK, rw - c0)
            p = k & 1
            if pending[p] is not None:
                pending[p].wait()
            dst = bufs[p] if c == CHUNK else bufs[p].at[pl.ds(0, c)]
            pltpu.async_copy(
                table_hbm.at[idx_v.at[pl.ds(_OFF_W[o] + c0, c)]],
                dst, gsems[p]).wait()
            pending[p] = pltpu.async_copy(
                dst, outs[o].at[pl.ds(base + c0, c)], ssems[p])
            k += 1
    for p in range(2):
        if pending[p] is not None:
            pending[p].wait()


def _sc_gather(table, idx_arrays):
    mesh = plsc.VectorSubcoreMesh(core_axis_name="c", subcore_axis_name="s")
    fn = functools.partial(
        pl.kernel, _sc_gather_body, mesh=mesh,
        out_type=[jax.ShapeDtypeStruct((s, D_MODEL), jnp.float32)
                  for s in _SIZES],
        scratch_types=[
            pltpu.VMEM((_IDX_VMEM,), jnp.int32),
            pltpu.VMEM((CHUNK, D_MODEL), jnp.float32),
            pltpu.VMEM((CHUNK, D_MODEL), jnp.float32),
            pltpu.SemaphoreType.DMA,
            pltpu.SemaphoreType.DMA,
            pltpu.SemaphoreType.DMA,
            pltpu.SemaphoreType.DMA,
        ],
    )()
    return fn(table, *idx_arrays)


def kernel(inputs_embeds, attention_mask):
    del inputs_embeds
    table = _build_table()
    idx_arrays = [jnp.asarray(ix) for ix in _IDX]
    embeds = _sc_gather(table, idx_arrays)
    cls_mask = _build_cls_mask()
    return tuple(embeds) + (attention_mask, cls_mask)


# speedup vs baseline: 1.9362x; 1.1476x over previous
import functools

import jax
import jax.numpy as jnp
import numpy as np
from jax import lax
from jax.experimental import pallas as pl
from jax.experimental.pallas import tpu as pltpu
from jax.experimental.pallas import tpu_sc as plsc

D_MODEL = 1024
SEQ_LEN = 2048
NUM_BLOCKS = 4
ZERO_OFFSET = 2 * SEQ_LEN
TBL_BASE = 2048
TBL_ROWS = 4160

NC, NS = 2, 16
NW = NC * NS
CHUNK = 32


def _pool_pos(pos, block_index):
    cls_pos = np.array([-(2 ** block_index) + 1], dtype=np.int64)
    return np.concatenate([cls_pos, pos[1:-1][::2]], axis=0)


def _rel_pos(pos, stride, pooled_pos=None, shift=1):
    if pooled_pos is None:
        pooled_pos = pos
    ref_point = int(pooled_pos[0]) - int(pos[0])
    num_remove = shift * len(pooled_pos)
    max_dist = ref_point + num_remove * stride
    min_dist = int(pooled_pos[0]) - int(pos[-1])
    return np.arange(max_dist, min_dist - 1, -stride, dtype=np.int64)


def _index_lists():
    pos = np.arange(0, SEQ_LEN, dtype=np.int64)
    pooled_pos = pos
    per_block = []
    for block_index in range(NUM_BLOCKS):
        pool_idx = None
        if block_index > 0:
            pooled_pos = _pool_pos(pos, block_index)
            stride = 2 ** (block_index - 1)
            pool_idx = _rel_pos(pos, stride, pooled_pos, shift=2) + ZERO_OFFSET
        pos = pooled_pos
        stride = 2 ** block_index
        np_idx = _rel_pos(pos, stride) + ZERO_OFFSET
        per_block.append((np_idx, pool_idx))
    flat = []
    for np_idx, pool_idx in per_block:
        flat.append(np_idx)
        if pool_idx is not None:
            flat.append(pool_idx)
    return [np.asarray(ix - TBL_BASE, dtype=np.int32) for ix in flat]


_IDX = _index_lists()
_SIZES = [len(ix) for ix in _IDX]
_ROWS_W = [s // NW for s in _SIZES]
_OFF_W = list(np.cumsum([0] + _ROWS_W))
_IDX_VMEM = int(np.ceil(_OFF_W[-1] / 16) * 16)
_OFF_G = list(np.cumsum([0] + _SIZES))
_IDX_ALL = np.concatenate(_IDX)


_TBL_BLK = 64
_HALF = D_MODEL // 2


def _table_body(out_ref):
    pid = pl.program_id(0)
    j = lax.broadcasted_iota(jnp.int32, (8, _HALF), 1).astype(jnp.float32)
    inv_freq = jnp.exp(j * jnp.float32(-np.log(10000.0) / _HALF))
    rel = (jnp.float32(pid * _TBL_BLK - TBL_BASE)
           + lax.broadcasted_iota(jnp.int32, (8, _HALF), 0).astype(jnp.float32))
    x = rel * inv_freq
    s, c = jnp.sin(x), jnp.cos(x)
    jd = lax.broadcasted_iota(jnp.int32, (1, _HALF), 1).astype(jnp.float32)
    d = 8.0 * jnp.exp(jd * jnp.float32(-np.log(10000.0) / _HALF))
    sd, cd = jnp.sin(d), jnp.cos(d)
    out_ref[0:8, :_HALF] = s
    out_ref[0:8, _HALF:] = c
    for k in range(1, _TBL_BLK // 8):
        s, c = s * cd + c * sd, c * cd - s * sd
        out_ref[8 * k:8 * k + 8, :_HALF] = s
        out_ref[8 * k:8 * k + 8, _HALF:] = c


def _build_table():
    return pl.pallas_call(
        _table_body,
        grid=(TBL_ROWS // _TBL_BLK,),
        out_specs=pl.BlockSpec((_TBL_BLK, D_MODEL), lambda i: (i, 0)),
        out_shape=jax.ShapeDtypeStruct((TBL_ROWS, D_MODEL), jnp.float32),
    )()


_CLS_BLK = 256


def _cls_body(out_ref):
    pid = pl.program_id(0)
    i = pid * _CLS_BLK + lax.broadcasted_iota(jnp.int32, (_CLS_BLK, SEQ_LEN), 0)
    j = lax.broadcasted_iota(jnp.int32, (_CLS_BLK, SEQ_LEN), 1)
    out_ref[...] = jnp.where((i > 0) & (j > 0), 1.0, 0.0).astype(jnp.float32)


def _build_cls_mask():
    return pl.pallas_call(
        _cls_body,
        grid=(SEQ_LEN // _CLS_BLK,),
        out_specs=pl.BlockSpec((_CLS_BLK, SEQ_LEN), lambda i: (i, 0)),
        out_shape=jax.ShapeDtypeStruct((SEQ_LEN, SEQ_LEN), jnp.float32),
    )()


def _sc_gather_body(table_hbm, idx_hbm, *refs):
    outs = refs[:7]
    idx_v, buf0, buf1, gsem0, gsem1, ssem0, ssem1 = refs[7:]
    bufs = (buf0, buf1)
    gsems = (gsem0, gsem1)
    ssems = (ssem0, ssem1)

    wid = lax.axis_index("s") * NC + lax.axis_index("c")

    for o in range(7):
        rw = _ROWS_W[o]
        pltpu.sync_copy(idx_hbm.at[pl.ds(_OFF_G[o] + wid * rw, rw)],
                        idx_v.at[pl.ds(_OFF_W[o], rw)])

    pending = [None, None]
    k = 0
    for o in range(7):
        rw = _ROWS_W[o]
        base = wid * rw
        for c0 in range(0, rw, CHUNK):
            c = min(CHUNK, rw - c0)
            p = k & 1
            if pending[p] is not None:
                pending[p].wait()
            dst = bufs[p] if c == CHUNK else bufs[p].at[pl.ds(0, c)]
            pltpu.async_copy(
                table_hbm.at[idx_v.at[pl.ds(_OFF_W[o] + c0, c)]],
                dst, gsems[p]).wait()
            pending[p] = pltpu.async_copy(
                dst, outs[o].at[pl.ds(base + c0, c)], ssems[p])
            k += 1
    for p in range(2):
        if pending[p] is not None:
            pending[p].wait()


def _sc_gather(table, idx_all):
    mesh = plsc.VectorSubcoreMesh(core_axis_name="c", subcore_axis_name="s")
    fn = functools.partial(
        pl.kernel, _sc_gather_body, mesh=mesh,
        out_type=[jax.ShapeDtypeStruct((s, D_MODEL), jnp.float32)
                  for s in _SIZES],
        scratch_types=[
            pltpu.VMEM((_IDX_VMEM,), jnp.int32),
            pltpu.VMEM((CHUNK, D_MODEL), jnp.float32),
            pltpu.VMEM((CHUNK, D_MODEL), jnp.float32),
            pltpu.SemaphoreType.DMA,
            pltpu.SemaphoreType.DMA,
            pltpu.SemaphoreType.DMA,
            pltpu.SemaphoreType.DMA,
        ],
    )()
    return fn(table, idx_all)


def kernel(inputs_embeds, attention_mask):
    del inputs_embeds
    table = _build_table()
    embeds = _sc_gather(table, jnp.asarray(_IDX_ALL))
    cls_mask = _build_cls_mask()
    return tuple(embeds) + (attention_mask, cls_mask)


# speedup vs baseline: 2.9949x; 1.5468x over previous
import functools

import jax
import jax.numpy as jnp
import numpy as np
from jax import lax
from jax.experimental import pallas as pl
from jax.experimental.pallas import tpu as pltpu
from jax.experimental.pallas import tpu_sc as plsc

D_MODEL = 1024
_HALF = D_MODEL // 2
SEQ_LEN = 2048
NUM_BLOCKS = 4
ZERO_OFFSET = 2 * SEQ_LEN
TBL_BASE = 2048
_LOGK = float(-np.log(10000.0) / _HALF)

NC, NS = 2, 16
NW = NC * NS
CHUNK = 32

HT_ROWS = 2560
_HT_BLK = 512
_SL_BLK = 512


def _pool_pos(pos, block_index):
    cls_pos = np.array([-(2 ** block_index) + 1], dtype=np.int64)
    return np.concatenate([cls_pos, pos[1:-1][::2]], axis=0)


def _rel_pos(pos, stride, pooled_pos=None, shift=1):
    if pooled_pos is None:
        pooled_pos = pos
    ref_point = int(pooled_pos[0]) - int(pos[0])
    num_remove = shift * len(pooled_pos)
    max_dist = ref_point + num_remove * stride
    min_dist = int(pooled_pos[0]) - int(pos[-1])
    return np.arange(max_dist, min_dist - 1, -stride, dtype=np.int64)


def _index_lists():
    pos = np.arange(0, SEQ_LEN, dtype=np.int64)
    pooled_pos = pos
    per_block = []
    for block_index in range(NUM_BLOCKS):
        pool_idx = None
        if block_index > 0:
            pooled_pos = _pool_pos(pos, block_index)
            stride = 2 ** (block_index - 1)
            pool_idx = _rel_pos(pos, stride, pooled_pos, shift=2) + ZERO_OFFSET
        pos = pooled_pos
        stride = 2 ** block_index
        np_idx = _rel_pos(pos, stride) + ZERO_OFFSET
        per_block.append((np_idx, pool_idx))
    flat = []
    for np_idx, pool_idx in per_block:
        flat.append(np_idx)
        if pool_idx is not None:
            flat.append(pool_idx)
    return [np.asarray(ix - TBL_BASE, dtype=np.int32) for ix in flat]


_IDX7 = _index_lists()
_SC_ORDER = (1, 3, 4, 5, 6)
for _o in _SC_ORDER:
    assert (_IDX7[_o] % 2 == 0).all()
_IDX_SC = [(_IDX7[_o] // 2).astype(np.int32) for _o in _SC_ORDER]
_SIZES = [len(ix) for ix in _IDX_SC]
_ROWS_W = [s // NW for s in _SIZES]
_OFF_W = [int(x) for x in np.cumsum([0] + _ROWS_W)]
_IDX_VMEM = int(np.ceil(_OFF_W[-1] / 16) * 16)
_OFF_G = [int(x) for x in np.cumsum([0] + _SIZES)]
_IDX_ALL = np.concatenate(_IDX_SC)


def _table_body(out_ref, b_ref):
    pid = pl.program_id(0)

    @pl.when(pid == 0)
    def _():
        i = lax.broadcasted_iota(jnp.int32, (64, _HALF), 0).astype(jnp.float32)
        j = lax.broadcasted_iota(jnp.int32, (64, _HALF), 1).astype(jnp.float32)
        x = (2.0 * i) * jnp.exp(j * jnp.float32(_LOGK))
        b_ref[:, :_HALF] = jnp.sin(x)
        b_ref[:, _HALF:] = jnp.cos(x)

    sin_b = b_ref[:, :_HALF]
    cos_b = b_ref[:, _HALF:]
    j1 = lax.broadcasted_iota(jnp.int32, (1, _HALF), 1).astype(jnp.float32)
    inv = jnp.exp(j1 * jnp.float32(_LOGK))
    for k in range(_HT_BLK // 64):
        rel0 = jnp.float32(2 * (pid * _HT_BLK + 64 * k) - 2048)
        xa = rel0 * inv
        sin_a, cos_a = jnp.sin(xa), jnp.cos(xa)
        out_ref[64 * k:64 * k + 64, :_HALF] = sin_a * cos_b + cos_a * sin_b
        out_ref[64 * k:64 * k + 64, _HALF:] = cos_a * cos_b - sin_a * sin_b


def _build_table():
    return pl.pallas_call(
        _table_body,
        grid=(HT_ROWS // _HT_BLK,),
        out_specs=pl.BlockSpec((_HT_BLK, D_MODEL), lambda i: (i, 0)),
        out_shape=jax.ShapeDtypeStruct((HT_ROWS, D_MODEL), jnp.float32),
        scratch_shapes=[pltpu.VMEM((64, D_MODEL), jnp.float32)],
    )()


def _slice_body(out_ref, b_ref, *, c0):
    pid = pl.program_id(0)

    @pl.when(pid == 0)
    def _():
        i = lax.broadcasted_iota(jnp.int32, (64, _HALF), 0).astype(jnp.float32)
        j = lax.broadcasted_iota(jnp.int32, (64, _HALF), 1).astype(jnp.float32)
        x = i * jnp.exp(j * jnp.float32(_LOGK))
        b_ref[:, :_HALF] = jnp.sin(x)
        b_ref[:, _HALF:] = jnp.cos(x)

    sin_b = b_ref[:, :_HALF]
    cos_b = b_ref[:, _HALF:]
    j1 = lax.broadcasted_iota(jnp.int32, (1, _HALF), 1).astype(jnp.float32)
    inv = jnp.exp(j1 * jnp.float32(_LOGK))
    for k in range(_SL_BLK // 64):
        rel0 = jnp.float32(c0 - (pid * _SL_BLK + 64 * k) - TBL_BASE)
        xa = rel0 * inv
        sin_a, cos_a = jnp.sin(xa), jnp.cos(xa)
        out_ref[64 * k:64 * k + 64, :_HALF] = sin_a * cos_b - cos_a * sin_b
        out_ref[64 * k:64 * k + 64, _HALF:] = cos_a * cos_b + sin_a * sin_b


def _build_slice(c0, rows):
    return pl.pallas_call(
        functools.partial(_slice_body, c0=c0),
        grid=(rows // _SL_BLK,),
        out_specs=pl.BlockSpec((_SL_BLK, D_MODEL), lambda i: (i, 0)),
        out_shape=jax.ShapeDtypeStruct((rows, D_MODEL), jnp.float32),
        scratch_shapes=[pltpu.VMEM((64, D_MODEL), jnp.float32)],
    )()


_CLS_BLK = 256


def _cls_body(out_ref):
    pid = pl.program_id(0)
    i = pid * _CLS_BLK + lax.broadcasted_iota(jnp.int32, (_CLS_BLK, SEQ_LEN), 0)
    j = lax.broadcasted_iota(jnp.int32, (_CLS_BLK, SEQ_LEN), 1)
    out_ref[...] = jnp.where((i > 0) & (j > 0), 1.0, 0.0).astype(jnp.float32)


def _build_cls_mask():
    return pl.pallas_call(
        _cls_body,
        grid=(SEQ_LEN // _CLS_BLK,),
        out_specs=pl.BlockSpec((_CLS_BLK, SEQ_LEN), lambda i: (i, 0)),
        out_shape=jax.ShapeDtypeStruct((SEQ_LEN, SEQ_LEN), jnp.float32),
    )()


def _sc_gather_body(table_hbm, idx_hbm, *refs):
    outs = refs[:5]
    idx_v, buf0, buf1, gsem0, gsem1, ssem0, ssem1 = refs[5:]
    bufs = (buf0, buf1)
    gsems = (gsem0, gsem1)
    ssems = (ssem0, ssem1)

    wid = lax.axis_index("s") * NC + lax.axis_index("c")

    for o in range(5):
        rw = _ROWS_W[o]
        pltpu.sync_copy(idx_hbm.at[pl.ds(_OFF_G[o] + wid * rw, rw)],
                        idx_v.at[pl.ds(_OFF_W[o], rw)])

    pending = [None, None]
    k = 0
    for o in range(5):
        rw = _ROWS_W[o]
        base = wid * rw
        for c0 in range(0, rw, CHUNK):
            c = min(CHUNK, rw - c0)
            p = k & 1
            if pending[p] is not None:
                pending[p].wait()
            dst = bufs[p] if c == CHUNK else bufs[p].at[pl.ds(0, c)]
            pltpu.async_copy(
                table_hbm.at[idx_v.at[pl.ds(_OFF_W[o] + c0, c)]],
                dst, gsems[p]).wait()
            pending[p] = pltpu.async_copy(
                dst, outs[o].at[pl.ds(base + c0, c)], ssems[p])
            k += 1
    for p in range(2):
        if pending[p] is not None:
            pending[p].wait()


def _sc_gather(table, idx_all):
    mesh = plsc.VectorSubcoreMesh(core_axis_name="c", subcore_axis_name="s")
    fn = functools.partial(
        pl.kernel, _sc_gather_body, mesh=mesh,
        out_type=[jax.ShapeDtypeStruct((s, D_MODEL), jnp.float32)
                  for s in _SIZES],
        scratch_types=[
            pltpu.VMEM((_IDX_VMEM,), jnp.int32),
            pltpu.VMEM((CHUNK, D_MODEL), jnp.float32),
            pltpu.VMEM((CHUNK, D_MODEL), jnp.float32),
            pltpu.SemaphoreType.DMA,
            pltpu.SemaphoreType.DMA,
            pltpu.SemaphoreType.DMA,
            pltpu.SemaphoreType.DMA,
        ],
    )()
    return fn(table, idx_all)


def kernel(inputs_embeds, attention_mask):
    del inputs_embeds
    table = _build_table()
    np1, np2, pool2, np3, pool3 = _sc_gather(table, jnp.asarray(_IDX_ALL))
    np0 = _build_slice(4096, 4096)
    pool1 = _build_slice(4095, 4096)
    cls_mask = _build_cls_mask()
    return (np0, np1, pool1, np2, pool2, np3, pool3, attention_mask, cls_mask)


# speedup vs baseline: 3.1395x; 1.0483x over previous
import functools

import jax
import jax.numpy as jnp
import numpy as np
from jax import lax
from jax.experimental import pallas as pl
from jax.experimental.pallas import tpu as pltpu
from jax.experimental.pallas import tpu_sc as plsc

D_MODEL = 1024
_HALF = D_MODEL // 2
SEQ_LEN = 2048
NUM_BLOCKS = 4
ZERO_OFFSET = 2 * SEQ_LEN
TBL_BASE = 2048
_LOGK = float(-np.log(10000.0) / _HALF)

NC, NS = 2, 16
NW = NC * NS
CHUNK = 32

HT_ROWS = 2560
_HT_BLK = 512
_SL_BLK = 512


def _pool_pos(pos, block_index):
    cls_pos = np.array([-(2 ** block_index) + 1], dtype=np.int64)
    return np.concatenate([cls_pos, pos[1:-1][::2]], axis=0)


def _rel_pos(pos, stride, pooled_pos=None, shift=1):
    if pooled_pos is None:
        pooled_pos = pos
    ref_point = int(pooled_pos[0]) - int(pos[0])
    num_remove = shift * len(pooled_pos)
    max_dist = ref_point + num_remove * stride
    min_dist = int(pooled_pos[0]) - int(pos[-1])
    return np.arange(max_dist, min_dist - 1, -stride, dtype=np.int64)


def _index_lists():
    pos = np.arange(0, SEQ_LEN, dtype=np.int64)
    pooled_pos = pos
    per_block = []
    for block_index in range(NUM_BLOCKS):
        pool_idx = None
        if block_index > 0:
            pooled_pos = _pool_pos(pos, block_index)
            stride = 2 ** (block_index - 1)
            pool_idx = _rel_pos(pos, stride, pooled_pos, shift=2) + ZERO_OFFSET
        pos = pooled_pos
        stride = 2 ** block_index
        np_idx = _rel_pos(pos, stride) + ZERO_OFFSET
        per_block.append((np_idx, pool_idx))
    flat = []
    for np_idx, pool_idx in per_block:
        flat.append(np_idx)
        if pool_idx is not None:
            flat.append(pool_idx)
    return [np.asarray(ix - TBL_BASE, dtype=np.int32) for ix in flat]


_IDX7 = _index_lists()
_SC_ORDER = (1, 3, 4, 5, 6)
for _o in _SC_ORDER:
    assert (_IDX7[_o] % 2 == 0).all()
_IDX_SC = [(_IDX7[_o] // 2).astype(np.int32) for _o in _SC_ORDER]
_SIZES = [len(ix) for ix in _IDX_SC]
_ROWS_W = [s // NW for s in _SIZES]
_OFF_W = [int(x) for x in np.cumsum([0] + _ROWS_W)]
_IDX_VMEM = int(np.ceil(_OFF_W[-1] / 16) * 16)
_OFF_G = [int(x) for x in np.cumsum([0] + _SIZES)]
_IDX_ALL = np.concatenate(_IDX_SC)


def _table_body(out_ref, b_ref):
    pid = pl.program_id(0)

    @pl.when(pid == 0)
    def _():
        i = lax.broadcasted_iota(jnp.int32, (64, _HALF), 0).astype(jnp.float32)
        j = lax.broadcasted_iota(jnp.int32, (64, _HALF), 1).astype(jnp.float32)
        x = (2.0 * i) * jnp.exp(j * jnp.float32(_LOGK))
        b_ref[:, :_HALF] = jnp.sin(x)
        b_ref[:, _HALF:] = jnp.cos(x)

    sin_b = b_ref[:, :_HALF]
    cos_b = b_ref[:, _HALF:]
    k8 = lax.broadcasted_iota(jnp.int32, (8, _HALF), 0).astype(jnp.float32)
    j8 = lax.broadcasted_iota(jnp.int32, (8, _HALF), 1).astype(jnp.float32)
    inv = jnp.exp(j8 * jnp.float32(_LOGK))
    xa = (jnp.float32(2 * pid * _HT_BLK - 2048) + 128.0 * k8) * inv
    sin_a8, cos_a8 = jnp.sin(xa), jnp.cos(xa)
    for k in range(_HT_BLK // 64):
        sin_a = sin_a8[k:k + 1, :]
        cos_a = cos_a8[k:k + 1, :]
        out_ref[64 * k:64 * k + 64, :_HALF] = sin_a * cos_b + cos_a * sin_b
        out_ref[64 * k:64 * k + 64, _HALF:] = cos_a * cos_b - sin_a * sin_b


def _build_table():
    return pl.pallas_call(
        _table_body,
        grid=(HT_ROWS // _HT_BLK,),
        out_specs=pl.BlockSpec((_HT_BLK, D_MODEL), lambda i: (i, 0)),
        out_shape=jax.ShapeDtypeStruct((HT_ROWS, D_MODEL), jnp.float32),
        scratch_shapes=[pltpu.VMEM((64, D_MODEL), jnp.float32)],
    )()


def _slice_body(out_ref, b_ref, *, c0):
    pid = pl.program_id(0)

    @pl.when(pid == 0)
    def _():
        i = lax.broadcasted_iota(jnp.int32, (64, _HALF), 0).astype(jnp.float32)
        j = lax.broadcasted_iota(jnp.int32, (64, _HALF), 1).astype(jnp.float32)
        x = i * jnp.exp(j * jnp.float32(_LOGK))
        b_ref[:, :_HALF] = jnp.sin(x)
        b_ref[:, _HALF:] = jnp.cos(x)

    sin_b = b_ref[:, :_HALF]
    cos_b = b_ref[:, _HALF:]
    k8 = lax.broadcasted_iota(jnp.int32, (8, _HALF), 0).astype(jnp.float32)
    j8 = lax.broadcasted_iota(jnp.int32, (8, _HALF), 1).astype(jnp.float32)
    inv = jnp.exp(j8 * jnp.float32(_LOGK))
    xa = (jnp.float32(c0 - pid * _SL_BLK - TBL_BASE) - 64.0 * k8) * inv
    sin_a8, cos_a8 = jnp.sin(xa), jnp.cos(xa)
    for k in range(_SL_BLK // 64):
        sin_a = sin_a8[k:k + 1, :]
        cos_a = cos_a8[k:k + 1, :]
        out_ref[64 * k:64 * k + 64, :_HALF] = sin_a * cos_b - cos_a * sin_b
        out_ref[64 * k:64 * k + 64, _HALF:] = cos_a * cos_b + sin_a * sin_b


def _build_slice(c0, rows):
    return pl.pallas_call(
        functools.partial(_slice_body, c0=c0),
        grid=(rows // _SL_BLK,),
        out_specs=pl.BlockSpec((_SL_BLK, D_MODEL), lambda i: (i, 0)),
        out_shape=jax.ShapeDtypeStruct((rows, D_MODEL), jnp.float32),
        scratch_shapes=[pltpu.VMEM((64, D_MODEL), jnp.float32)],
    )()


_CLS_BLK = 256


def _cls_body(out_ref):
    pid = pl.program_id(0)
    i = pid * _CLS_BLK + lax.broadcasted_iota(jnp.int32, (_CLS_BLK, SEQ_LEN), 0)
    j = lax.broadcasted_iota(jnp.int32, (_CLS_BLK, SEQ_LEN), 1)
    out_ref[...] = jnp.where((i > 0) & (j > 0), 1.0, 0.0).astype(jnp.float32)


def _build_cls_mask():
    return pl.pallas_call(
        _cls_body,
        grid=(SEQ_LEN // _CLS_BLK,),
        out_specs=pl.BlockSpec((_CLS_BLK, SEQ_LEN), lambda i: (i, 0)),
        out_shape=jax.ShapeDtypeStruct((SEQ_LEN, SEQ_LEN), jnp.float32),
    )()


def _sc_gather_body(table_hbm, idx_hbm, *refs):
    outs = refs[:5]
    idx_v, buf0, buf1, gsem0, gsem1, ssem0, ssem1 = refs[5:]
    bufs = (buf0, buf1)
    gsems = (gsem0, gsem1)
    ssems = (ssem0, ssem1)

    wid = lax.axis_index("s") * NC + lax.axis_index("c")

    for o in range(5):
        rw = _ROWS_W[o]
        pltpu.sync_copy(idx_hbm.at[pl.ds(_OFF_G[o] + wid * rw, rw)],
                        idx_v.at[pl.ds(_OFF_W[o], rw)])

    pending = [None, None]
    k = 0
    for o in range(5):
        rw = _ROWS_W[o]
        base = wid * rw
        for c0 in range(0, rw, CHUNK):
            c = min(CHUNK, rw - c0)
            p = k & 1
            if pending[p] is not None:
                pending[p].wait()
            dst = bufs[p] if c == CHUNK else bufs[p].at[pl.ds(0, c)]
            pltpu.async_copy(
                table_hbm.at[idx_v.at[pl.ds(_OFF_W[o] + c0, c)]],
                dst, gsems[p]).wait()
            pending[p] = pltpu.async_copy(
                dst, outs[o].at[pl.ds(base + c0, c)], ssems[p])
            k += 1
    for p in range(2):
        if pending[p] is not None:
            pending[p].wait()


def _sc_gather(table, idx_all):
    mesh = plsc.VectorSubcoreMesh(core_axis_name="c", subcore_axis_name="s")
    fn = functools.partial(
        pl.kernel, _sc_gather_body, mesh=mesh,
        out_type=[jax.ShapeDtypeStruct((s, D_MODEL), jnp.float32)
                  for s in _SIZES],
        scratch_types=[
            pltpu.VMEM((_IDX_VMEM,), jnp.int32),
            pltpu.VMEM((CHUNK, D_MODEL), jnp.float32),
            pltpu.VMEM((CHUNK, D_MODEL), jnp.float32),
            pltpu.SemaphoreType.DMA,
            pltpu.SemaphoreType.DMA,
            pltpu.SemaphoreType.DMA,
            pltpu.SemaphoreType.DMA,
        ],
    )()
    return fn(table, idx_all)


def kernel(inputs_embeds, attention_mask):
    del inputs_embeds
    table = _build_table()
    np1, np2, pool2, np3, pool3 = _sc_gather(table, jnp.asarray(_IDX_ALL))
    np0 = _build_slice(4096, 4096)
    pool1 = _build_slice(4095, 4096)
    cls_mask = _build_cls_mask()
    return (np0, np1, pool1, np2, pool2, np3, pool3, attention_mask, cls_mask)


# speedup vs baseline: 3.2695x; 1.0414x over previous
import functools

import jax
import jax.numpy as jnp
import numpy as np
from jax import lax
from jax.experimental import pallas as pl
from jax.experimental.pallas import tpu as pltpu
from jax.experimental.pallas import tpu_sc as plsc

D_MODEL = 1024
_HALF = D_MODEL // 2
SEQ_LEN = 2048
NUM_BLOCKS = 4
ZERO_OFFSET = 2 * SEQ_LEN
TBL_BASE = 2048
_LOGK = float(-np.log(10000.0) / _HALF)

NC, NS = 2, 16
NW = NC * NS
CHUNK = 32

HT_ROWS = 2112
_HT_BLK = 704
_SL_BLK = 512


def _pool_pos(pos, block_index):
    cls_pos = np.array([-(2 ** block_index) + 1], dtype=np.int64)
    return np.concatenate([cls_pos, pos[1:-1][::2]], axis=0)


def _rel_pos(pos, stride, pooled_pos=None, shift=1):
    if pooled_pos is None:
        pooled_pos = pos
    ref_point = int(pooled_pos[0]) - int(pos[0])
    num_remove = shift * len(pooled_pos)
    max_dist = ref_point + num_remove * stride
    min_dist = int(pooled_pos[0]) - int(pos[-1])
    return np.arange(max_dist, min_dist - 1, -stride, dtype=np.int64)


def _index_lists():
    pos = np.arange(0, SEQ_LEN, dtype=np.int64)
    pooled_pos = pos
    per_block = []
    for block_index in range(NUM_BLOCKS):
        pool_idx = None
        if block_index > 0:
            pooled_pos = _pool_pos(pos, block_index)
            stride = 2 ** (block_index - 1)
            pool_idx = _rel_pos(pos, stride, pooled_pos, shift=2) + ZERO_OFFSET
        pos = pooled_pos
        stride = 2 ** block_index
        np_idx = _rel_pos(pos, stride) + ZERO_OFFSET
        per_block.append((np_idx, pool_idx))
    flat = []
    for np_idx, pool_idx in per_block:
        flat.append(np_idx)
        if pool_idx is not None:
            flat.append(pool_idx)
    return [np.asarray(ix - TBL_BASE, dtype=np.int32) for ix in flat]


_IDX7 = _index_lists()
_SC_ORDER = (1, 3, 4, 5, 6)
for _o in _SC_ORDER:
    assert (_IDX7[_o] % 2 == 0).all()
_IDX_SC = [(_IDX7[_o] // 2).astype(np.int32) for _o in _SC_ORDER]
_SIZES = [len(ix) for ix in _IDX_SC]
_ROWS_W = [s // NW for s in _SIZES]
_OFF_W = [int(x) for x in np.cumsum([0] + _ROWS_W)]
_IDX_VMEM = int(np.ceil(_OFF_W[-1] / 16) * 16)
_OFF_G = [int(x) for x in np.cumsum([0] + _SIZES)]
_IDX_ALL = np.concatenate(_IDX_SC)


def _table_body(out_ref, b_ref):
    pid = pl.program_id(0)

    @pl.when(pid == 0)
    def _():
        i = lax.broadcasted_iota(jnp.int32, (64, _HALF), 0).astype(jnp.float32)
        j = lax.broadcasted_iota(jnp.int32, (64, _HALF), 1).astype(jnp.float32)
        x = (2.0 * i) * jnp.exp(j * jnp.float32(_LOGK))
        b_ref[:, :_HALF] = jnp.sin(x)
        b_ref[:, _HALF:] = jnp.cos(x)

    sin_b = b_ref[:, :_HALF]
    cos_b = b_ref[:, _HALF:]
    k16 = lax.broadcasted_iota(jnp.int32, (16, _HALF), 0).astype(jnp.float32)
    j16 = lax.broadcasted_iota(jnp.int32, (16, _HALF), 1).astype(jnp.float32)
    inv = jnp.exp(j16 * jnp.float32(_LOGK))
    xa = (jnp.float32(2 * pid * _HT_BLK - 2048) + 128.0 * k16) * inv
    sin_a16, cos_a16 = jnp.sin(xa), jnp.cos(xa)
    for k in range(_HT_BLK // 64):
        sin_a = sin_a16[k:k + 1, :]
        cos_a = cos_a16[k:k + 1, :]
        out_ref[64 * k:64 * k + 64, :_HALF] = sin_a * cos_b + cos_a * sin_b
        out_ref[64 * k:64 * k + 64, _HALF:] = cos_a * cos_b - sin_a * sin_b


def _build_table():
    return pl.pallas_call(
        _table_body,
        grid=(HT_ROWS // _HT_BLK,),
        out_specs=pl.BlockSpec((_HT_BLK, D_MODEL), lambda i: (i, 0)),
        out_shape=jax.ShapeDtypeStruct((HT_ROWS, D_MODEL), jnp.float32),
        scratch_shapes=[pltpu.VMEM((64, D_MODEL), jnp.float32)],
    )()


def _slice_body(np0_ref, pool1_ref, b_ref):
    pid = pl.program_id(0)

    @pl.when(pid == 0)
    def _():
        i = lax.broadcasted_iota(jnp.int32, (64, _HALF), 0).astype(jnp.float32)
        j = lax.broadcasted_iota(jnp.int32, (64, _HALF), 1).astype(jnp.float32)
        x = i * jnp.exp(j * jnp.float32(_LOGK))
        b_ref[:, :_HALF] = jnp.sin(x)
        b_ref[:, _HALF:] = jnp.cos(x)

    sin_b = b_ref[:, :_HALF]
    cos_b = b_ref[:, _HALF:]
    r16 = lax.broadcasted_iota(jnp.int32, (16, _HALF), 0)
    j16 = lax.broadcasted_iota(jnp.int32, (16, _HALF), 1).astype(jnp.float32)
    inv = jnp.exp(j16 * jnp.float32(_LOGK))
    k16 = (r16 % 8).astype(jnp.float32)
    c16 = jnp.where(r16 < 8, 4096.0, 4095.0)
    xa = (c16 - jnp.float32(pid * _SL_BLK + TBL_BASE) - 64.0 * k16) * inv
    sin_a16, cos_a16 = jnp.sin(xa), jnp.cos(xa)
    for k in range(_SL_BLK // 64):
        for out_ref, row in ((np0_ref, k), (pool1_ref, k + 8)):
            sin_a = sin_a16[row:row + 1, :]
            cos_a = cos_a16[row:row + 1, :]
            out_ref[64 * k:64 * k + 64, :_HALF] = sin_a * cos_b - cos_a * sin_b
            out_ref[64 * k:64 * k + 64, _HALF:] = cos_a * cos_b + sin_a * sin_b


def _build_slices():
    return pl.pallas_call(
        _slice_body,
        grid=(4096 // _SL_BLK,),
        out_specs=[pl.BlockSpec((_SL_BLK, D_MODEL), lambda i: (i, 0))] * 2,
        out_shape=[jax.ShapeDtypeStruct((4096, D_MODEL), jnp.float32)] * 2,
        scratch_shapes=[pltpu.VMEM((64, D_MODEL), jnp.float32)],
    )()


_CLS_BLK = 256


def _cls_body(out_ref):
    pid = pl.program_id(0)
    i = pid * _CLS_BLK + lax.broadcasted_iota(jnp.int32, (_CLS_BLK, SEQ_LEN), 0)
    j = lax.broadcasted_iota(jnp.int32, (_CLS_BLK, SEQ_LEN), 1)
    out_ref[...] = jnp.where((i > 0) & (j > 0), 1.0, 0.0).astype(jnp.float32)


def _build_cls_mask():
    return pl.pallas_call(
        _cls_body,
        grid=(SEQ_LEN // _CLS_BLK,),
        out_specs=pl.BlockSpec((_CLS_BLK, SEQ_LEN), lambda i: (i, 0)),
        out_shape=jax.ShapeDtypeStruct((SEQ_LEN, SEQ_LEN), jnp.float32),
    )()


def _sc_gather_body(table_hbm, idx_hbm, *refs):
    outs = refs[:5]
    idx_v, buf0, buf1, gsem0, gsem1, ssem0, ssem1 = refs[5:]
    bufs = (buf0, buf1)
    gsems = (gsem0, gsem1)
    ssems = (ssem0, ssem1)

    wid = lax.axis_index("s") * NC + lax.axis_index("c")

    for o in range(5):
        rw = _ROWS_W[o]
        pltpu.sync_copy(idx_hbm.at[pl.ds(_OFF_G[o] + wid * rw, rw)],
                        idx_v.at[pl.ds(_OFF_W[o], rw)])

    pending = [None, None]
    k = 0
    for o in range(5):
        rw = _ROWS_W[o]
        base = wid * rw
        for c0 in range(0, rw, CHUNK):
            c = min(CHUNK, rw - c0)
            p = k & 1
            if pending[p] is not None:
                pending[p].wait()
            dst = bufs[p] if c == CHUNK else bufs[p].at[pl.ds(0, c)]
            pltpu.async_copy(
                table_hbm.at[idx_v.at[pl.ds(_OFF_W[o] + c0, c)]],
                dst, gsems[p]).wait()
            pending[p] = pltpu.async_copy(
                dst, outs[o].at[pl.ds(base + c0, c)], ssems[p])
            k += 1
    for p in range(2):
        if pending[p] is not None:
            pending[p].wait()


def _sc_gather(table, idx_all):
    mesh = plsc.VectorSubcoreMesh(core_axis_name="c", subcore_axis_name="s")
    fn = functools.partial(
        pl.kernel, _sc_gather_body, mesh=mesh,
        out_type=[jax.ShapeDtypeStruct((s, D_MODEL), jnp.float32)
                  for s in _SIZES],
        scratch_types=[
            pltpu.VMEM((_IDX_VMEM,), jnp.int32),
            pltpu.VMEM((CHUNK, D_MODEL), jnp.float32),
            pltpu.VMEM((CHUNK, D_MODEL), jnp.float32),
            pltpu.SemaphoreType.DMA,
            pltpu.SemaphoreType.DMA,
            pltpu.SemaphoreType.DMA,
            pltpu.SemaphoreType.DMA,
        ],
    )()
    return fn(table, idx_all)


def kernel(inputs_embeds, attention_mask):
    del inputs_embeds
    table = _build_table()
    np1, np2, pool2, np3, pool3 = _sc_gather(table, jnp.asarray(_IDX_ALL))
    np0, pool1 = _build_slices()
    cls_mask = _build_cls_mask()
    return (np0, np1, pool1, np2, pool2, np3, pool3, attention_mask, cls_mask)
